# Initial kernel scaffold; baseline (speedup 1.0000x reference)
import jax
import jax.numpy as jnp
from jax.experimental import pallas as pl


def kernel(fc_log, genotypes, expression_obs, variantxgene_to_gene, local_variant_to_local_variantxgene_selector, variantxgene_to_local_gene, lib, baseline_log, dispersion_log):
    raise NotImplementedError("write your pallas kernel here")


# speedup vs baseline: 8.2442x; 8.2442x over previous
import functools

import jax
import jax.numpy as jnp
from jax import lax
from jax.experimental import pallas as pl
from jax.experimental.pallas import tpu as pltpu
from jax.experimental.pallas import tpu_sc as plsc

_NC = 2
_NS = 16
_NW = _NC * _NS


def _sc_gather(table_b, idx_b, table_g, idx_g):
    n_idx = idx_b.shape[0]
    bpw = n_idx // _NW
    db = table_b.shape[1]
    dg = table_g.shape[1]
    mesh = plsc.VectorSubcoreMesh(core_axis_name="c", subcore_axis_name="s")

    @functools.partial(
        pl.kernel,
        mesh=mesh,
        out_type=[
            jax.ShapeDtypeStruct((n_idx, db), jnp.float32),
            jax.ShapeDtypeStruct((n_idx, dg), jnp.float32),
        ],
        scratch_types=[
            pltpu.VMEM((bpw,), jnp.int32),
            pltpu.VMEM((bpw, db), jnp.float32),
            pltpu.VMEM((bpw,), jnp.int32),
            pltpu.VMEM((bpw, dg), jnp.float32),
            pltpu.SemaphoreType.DMA,
            pltpu.SemaphoreType.DMA,
        ],
        compiler_params=pltpu.CompilerParams(use_tc_tiling_on_sc=False),
    )
    def gather_kernel(tb, ib, tg, ig, ob, og, ibv, rbv, igv, rgv, semb, semg):
        wid = lax.axis_index("s") * _NC + lax.axis_index("c")
        base = wid * bpw
        pltpu.sync_copy(ib.at[pl.ds(base, bpw)], ibv)
        pltpu.sync_copy(ig.at[pl.ds(base, bpw)], igv)
        cb = pltpu.async_copy(tb.at[ibv], rbv, semb)
        cg = pltpu.async_copy(tg.at[igv], rgv, semg)
        cb.wait()
        cg.wait()
        pltpu.sync_copy(rbv, ob.at[pl.ds(base, bpw)])
        pltpu.sync_copy(rgv, og.at[pl.ds(base, bpw)])

    return gather_kernel(table_b, idx_b, table_g, idx_g)


def _tc_body(bt_ref, gt_ref, fc_ref, lib_ref, o_ref, *, n_clusters):
    b = bt_ref[...].T[:n_clusters, :]
    g = gt_ref[...].T
    fc = fc_ref[...]
    lib = lib_ref[...]
    x = b[None, :, :] + g[:, None, :] * fc[None, :, :]
    o_ref[...] = jnp.exp(x) * lib[:, :, None]


def kernel(fc_log, genotypes, expression_obs, variantxgene_to_gene,
           local_variant_to_local_variantxgene_selector, variantxgene_to_local_gene,
           lib, baseline_log, dispersion_log):
    n_clusters, n_vxg = fc_log.shape
    n_donors = genotypes.shape[0]
    cpad = 32

    table_b = jnp.pad(baseline_log.T, ((0, 0), (0, cpad - n_clusters)))
    table_g = genotypes.T
    bt, gt = _sc_gather(table_b, variantxgene_to_gene,
                        table_g, local_variant_to_local_variantxgene_selector)

    vb = 512
    out = pl.pallas_call(
        functools.partial(_tc_body, n_clusters=n_clusters),
        grid=(n_vxg // vb,),
        in_specs=[
            pl.BlockSpec((vb, cpad), lambda i: (i, 0)),
            pl.BlockSpec((vb, n_donors), lambda i: (i, 0)),
            pl.BlockSpec((n_clusters, vb), lambda i: (0, i)),
            pl.BlockSpec((n_donors, n_clusters), lambda i: (0, 0)),
        ],
        out_specs=pl.BlockSpec((n_donors, n_clusters, vb), lambda i: (0, 0, i)),
        out_shape=jax.ShapeDtypeStruct((n_donors, n_clusters, n_vxg), jnp.float32),
    )(bt, gt, fc_log, lib)
    return out
